# baseline (speedup 1.0000x reference)
import functools

import jax
import jax.numpy as jnp
import numpy as np
from jax import lax
from jax.experimental import pallas as pl
from jax.experimental.pallas import tpu as pltpu
from jax.experimental.pallas import tpu_sc as plsc

_VOCAB = 1000000
_DIM = 64
_BASE = 10000.0

_NC = 2
_NS = 16
_NW = _NC * _NS

_B = 4096
_L = 200
_N = _B * _L
_PER_W = _N // _NW
_IW = 128
_ROWS_PER_W = _PER_W // _IW
_G = 5
_GROUP_ROWS = _G * _IW
_NGROUPS = _ROWS_PER_W // _G


def _make_pe():
    pos = jnp.arange(_L, dtype=jnp.float32)[:, None]
    div = jnp.exp(
        jnp.arange(0, _DIM, 2, dtype=jnp.float32) * (-jnp.log(_BASE) / _DIM)
    )
    pe = jnp.zeros((_L, _DIM), dtype=jnp.float32)
    pe = pe.at[:, 0::2].set(jnp.sin(pos * div))
    pe = pe.at[:, 1::2].set(jnp.cos(pos * div))
    return pe


def _sc_body(idx_hbm, table_hbm, pe_hbm, out_hbm, idx_v, pe_v, rows_v, gsem):
    c = lax.axis_index("c")
    s = lax.axis_index("s")
    wid = s * _NC + c
    row_base = wid * _ROWS_PER_W

    pltpu.sync_copy(idx_hbm.at[pl.ds(row_base, _ROWS_PER_W)], idx_v)
    pltpu.sync_copy(pe_hbm, pe_v)

    def group(g, _):
        copies = []
        for j in range(_G):
            cp = pltpu.async_copy(
                table_hbm.at[idx_v.at[g * _G + j]],
                rows_v.at[pl.ds(j * _IW, _IW)],
                gsem,
            )
            copies.append(cp)
        for cp in copies:
            cp.wait()

        off = (g * _GROUP_ROWS) % _L

        def add_row(i, _):
            jrow = (off + i) % _L
            for k in range(_DIM // 16):
                plsc.addupdate(
                    rows_v.at[i, pl.ds(k * 16, 16)],
                    pe_v[jrow, pl.ds(k * 16, 16)],
                )
            return 0

        lax.fori_loop(0, _GROUP_ROWS, add_row, 0)

        pltpu.sync_copy(
            rows_v,
            out_hbm.at[pl.ds(wid * _PER_W + g * _GROUP_ROWS, _GROUP_ROWS)],
        )
        return 0

    lax.fori_loop(0, _NGROUPS, group, 0)


@jax.jit
def kernel(x, table):
    pe = _make_pe()
    idx = x.reshape(_ROWS_PER_W * _NW, _IW).astype(jnp.int32)

    mesh = plsc.VectorSubcoreMesh(core_axis_name="c", subcore_axis_name="s")
    out = pl.kernel(
        _sc_body,
        out_type=jax.ShapeDtypeStruct((_N, _DIM), jnp.float32),
        mesh=mesh,
        scratch_types=[
            pltpu.VMEM((_ROWS_PER_W, _IW), jnp.int32),
            pltpu.VMEM((_L, _DIM), jnp.float32),
            pltpu.VMEM((_GROUP_ROWS, _DIM), jnp.float32),
            pltpu.SemaphoreType.DMA,
        ],
        compiler_params=pltpu.CompilerParams(use_tc_tiling_on_sc=False),
    )(idx, table, pe)
    return out.reshape(_B, _L, _DIM)
